# Initial kernel scaffold; baseline (speedup 1.0000x reference)
import jax
import jax.numpy as jnp
from jax.experimental import pallas as pl


def kernel(Q, K, V, rng):
    raise NotImplementedError("write your pallas kernel here")


# speedup vs baseline: 1.5837x; 1.5837x over previous
import math

import jax
import jax.numpy as jnp
from jax import lax
from jax.experimental import pallas as pl

_C = 10
_ROW_BLK = 128


def _attn_body(q_ref, k_ref, v_ref, w_ref, out_ref):
    m = q_ref.shape[1]
    n = k_ref.shape[1]
    d = q_ref.shape[2]
    u = int(_C * math.log(m))
    u_pad = ((u + 7) // 8) * 8
    nr = m // _ROW_BLK
    neg = jnp.float32(-3e38)

    k_all = k_ref[0]
    w2 = w_ref[0]
    msk = w2 > 0.0

    m_rows = []
    for r in range(nr):
        qb = q_ref[0, r * _ROW_BLK:(r + 1) * _ROW_BLK, :]
        s = lax.dot_general(qb, k_all, (((1,), (1,)), ((), ())),
                            preferred_element_type=jnp.float32)
        mmax = jnp.max(jnp.where(msk, s, neg), axis=1)
        msum = jnp.sum(s * w2, axis=1)
        m_rows.append(mmax - msum)
    M = jnp.concatenate(m_rows).reshape(nr, _ROW_BLK)

    flat = (lax.broadcasted_iota(jnp.int32, (nr, _ROW_BLK), 0) * _ROW_BLK
            + lax.broadcasted_iota(jnp.int32, (nr, _ROW_BLK), 1))
    rid = lax.broadcasted_iota(jnp.int32, (u_pad, 1), 0)

    def step(t, carry):
        mw, iv = carry
        mx = jnp.max(mw)
        loc = jnp.min(jnp.where(mw == mx, flat, jnp.int32(2 ** 30)))
        iv = jnp.where(rid == t, loc, iv)
        mw = jnp.where(flat == loc, neg, mw)
        return mw, iv

    _, iv = lax.fori_loop(
        0, u, step, (M, jnp.full((u_pad, 1), -1, jnp.int32)))

    col = lax.broadcasted_iota(jnp.int32, (u_pad, n), 1)
    onehot = (col == iv).astype(jnp.float32)
    qbar = lax.dot_general(onehot, q_ref[0], (((1,), (0,)), ((), ())),
                           precision=lax.Precision.HIGHEST,
                           preferred_element_type=jnp.float32)

    s2 = lax.dot_general(qbar, k_all, (((1,), (1,)), ((), ())),
                         preferred_element_type=jnp.float32)
    s2 = s2 * jnp.float32(1.0 / math.sqrt(d))
    s2 = s2 - jnp.max(s2, axis=1, keepdims=True)
    e = jnp.exp(s2)
    p = e / jnp.sum(e, axis=1, keepdims=True)
    s1 = lax.dot_general(p, v_ref[0], (((1,), (0,)), ((), ())),
                         preferred_element_type=jnp.float32)

    ones_u = jnp.ones((u_pad, 1), jnp.float32)
    for r in range(nr):
        ob = onehot[:, r * _ROW_BLK:(r + 1) * _ROW_BLK]
        cb = lax.dot_general(ob, s1, (((0,), (0,)), ((), ())),
                             precision=lax.Precision.HIGHEST,
                             preferred_element_type=jnp.float32)
        indb = lax.dot_general(ob, ones_u, (((0,), (0,)), ((), ())),
                               precision=lax.Precision.HIGHEST,
                               preferred_element_type=jnp.float32)
        vb = v_ref[0, r * _ROW_BLK:(r + 1) * _ROW_BLK, :]
        vmb = jnp.sum(vb, axis=1, keepdims=True) * jnp.float32(1.0 / d)
        out_ref[0, r * _ROW_BLK:(r + 1) * _ROW_BLK, :] = (
            cb + (1.0 - indb) * vmb)


def kernel(Q, K, V, rng):
    B, m, d = Q.shape
    n = K.shape[1]
    U = int(m * math.log(n))

    rng_batch = jax.random.split(rng, B)
    idx = jax.vmap(
        lambda r: jax.random.choice(r, n, shape=(U,), replace=True))(rng_batch)
    counts = jax.vmap(lambda ix: jnp.bincount(ix, length=n))(idx)
    w = (counts.astype(jnp.float32) * jnp.float32(1.0 / U)).reshape(B, 1, n)

    out = pl.pallas_call(
        _attn_body,
        grid=(B,),
        in_specs=[
            pl.BlockSpec((1, m, d), lambda b: (b, 0, 0)),
            pl.BlockSpec((1, n, d), lambda b: (b, 0, 0)),
            pl.BlockSpec((1, n, d), lambda b: (b, 0, 0)),
            pl.BlockSpec((1, 1, n), lambda b: (b, 0, 0)),
        ],
        out_specs=pl.BlockSpec((1, m, d), lambda b: (b, 0, 0)),
        out_shape=jax.ShapeDtypeStruct((B, m, d), jnp.float32),
    )(Q, K, V, w)
    return out


# speedup vs baseline: 1.8876x; 1.1919x over previous
import math

import jax
import jax.numpy as jnp
from jax import lax
from jax.experimental import pallas as pl

_C = 10
_ROW_BLK = 128


def _attn_body(q_ref, k_ref, v_ref, w_ref, out_ref):
    m = q_ref.shape[1]
    n = k_ref.shape[1]
    d = q_ref.shape[2]
    u = int(_C * math.log(m))
    u_pad = ((u + 7) // 8) * 8
    nr = m // _ROW_BLK
    neg = jnp.float32(-3e38)

    k_all = k_ref[0]
    w2 = w_ref[0]
    msk = w2 > 0.0

    m_rows = []
    for r in range(nr):
        qb = q_ref[0, r * _ROW_BLK:(r + 1) * _ROW_BLK, :]
        s = lax.dot_general(qb, k_all, (((1,), (1,)), ((), ())),
                            preferred_element_type=jnp.float32)
        mmax = jnp.max(jnp.where(msk, s, neg), axis=1)
        msum = jnp.sum(s * w2, axis=1)
        m_rows.append(mmax - msum)
    m_row = jnp.concatenate(m_rows)[None, :]
    m_col = m_row.reshape(m, 1)

    j_row = lax.broadcasted_iota(jnp.int32, (1, m), 1)
    cnt_cols = []
    for r in range(nr):
        a = m_col[r * _ROW_BLK:(r + 1) * _ROW_BLK, :]
        i_col = (lax.broadcasted_iota(jnp.int32, (_ROW_BLK, 1), 0)
                 + jnp.int32(r * _ROW_BLK))
        beats = (m_row > a) | ((m_row == a) & (j_row < i_col))
        cnt_cols.append(jnp.sum(jnp.where(beats, 1.0, 0.0),
                                axis=1, keepdims=True))
    cnt_col = jnp.concatenate(cnt_cols, axis=0)
    cnt_row = cnt_col.reshape(1, m)

    rid = lax.broadcasted_iota(jnp.int32, (u_pad, 1), 0).astype(jnp.float32)
    onehot = jnp.where((cnt_row == rid) & (rid < u), 1.0, 0.0)
    qbar = lax.dot_general(onehot, q_ref[0], (((1,), (0,)), ((), ())),
                           precision=lax.Precision.HIGHEST,
                           preferred_element_type=jnp.float32)

    s2 = lax.dot_general(qbar, k_all, (((1,), (1,)), ((), ())),
                         preferred_element_type=jnp.float32)
    s2 = s2 * jnp.float32(1.0 / math.sqrt(d))
    s2 = s2 - jnp.max(s2, axis=1, keepdims=True)
    e = jnp.exp(s2)
    p = e / jnp.sum(e, axis=1, keepdims=True)
    s1 = lax.dot_general(p, v_ref[0], (((1,), (0,)), ((), ())),
                         preferred_element_type=jnp.float32)

    for r in range(nr):
        ob = onehot[:, r * _ROW_BLK:(r + 1) * _ROW_BLK]
        cb = lax.dot_general(ob, s1, (((0,), (0,)), ((), ())),
                             precision=lax.Precision.HIGHEST,
                             preferred_element_type=jnp.float32)
        memb = cnt_cols[r] < u
        vb = v_ref[0, r * _ROW_BLK:(r + 1) * _ROW_BLK, :]
        vmb = jnp.sum(vb, axis=1, keepdims=True) * jnp.float32(1.0 / d)
        out_ref[0, r * _ROW_BLK:(r + 1) * _ROW_BLK, :] = (
            jnp.where(memb, cb, vmb))


def kernel(Q, K, V, rng):
    B, m, d = Q.shape
    n = K.shape[1]
    U = int(m * math.log(n))

    rng_batch = jax.random.split(rng, B)
    idx = jax.vmap(
        lambda r: jax.random.choice(r, n, shape=(U,), replace=True))(rng_batch)
    counts = jax.vmap(lambda ix: jnp.bincount(ix, length=n))(idx)
    w = (counts.astype(jnp.float32) * jnp.float32(1.0 / U)).reshape(B, 1, n)

    out = pl.pallas_call(
        _attn_body,
        grid=(B,),
        in_specs=[
            pl.BlockSpec((1, m, d), lambda b: (b, 0, 0)),
            pl.BlockSpec((1, n, d), lambda b: (b, 0, 0)),
            pl.BlockSpec((1, n, d), lambda b: (b, 0, 0)),
            pl.BlockSpec((1, 1, n), lambda b: (b, 0, 0)),
        ],
        out_specs=pl.BlockSpec((1, m, d), lambda b: (b, 0, 0)),
        out_shape=jax.ShapeDtypeStruct((B, m, d), jnp.float32),
    )(Q, K, V, w)
    return out


# speedup vs baseline: 4.4104x; 2.3365x over previous
import functools
import math

import jax
import jax.numpy as jnp
from jax import lax
from jax.experimental import pallas as pl
from jax.experimental.pallas import tpu as pltpu
from jax.experimental.pallas import tpu_sc as plsc

_C = 10
_ROW_BLK = 128
_NSUB = 16


def _sc_hist_body(idx_hbm, out_hbm, idx_v, loc_v):
    c = lax.axis_index("c")
    s = lax.axis_index("s")
    n = loc_v.shape[0]
    chunk = idx_v.shape[0]

    def zero_body(i, carry):
        loc_v[pl.ds(i * 16, 16)] = jnp.zeros((16,), jnp.float32)
        return carry

    lax.fori_loop(0, n // 16, zero_body, 0)

    pltpu.sync_copy(idx_hbm.at[pl.ds(c * (chunk * _NSUB) + s * chunk, chunk)],
                    idx_v)

    ones = jnp.ones((16,), jnp.float32)

    def body(i, carry):
        iv = idx_v[pl.ds(i * 16, 16)]
        plsc.addupdate_scatter(loc_v, [jnp.maximum(iv, 0)], ones,
                               mask=iv >= 0)
        return carry

    lax.fori_loop(0, chunk // 16, body, 0)

    pltpu.sync_copy(loc_v, out_hbm.at[pl.ds((c * _NSUB + s) * n, n)])


def _sc_histogram(idx_flat, B, n):
    UP = idx_flat.shape[0] // B
    mesh = plsc.VectorSubcoreMesh(core_axis_name="c", subcore_axis_name="s")
    out = pl.kernel(
        _sc_hist_body,
        out_type=jax.ShapeDtypeStruct((B * _NSUB * n,), jnp.float32),
        mesh=mesh,
        scratch_types=[
            pltpu.VMEM((UP // _NSUB,), jnp.int32),
            pltpu.VMEM((n,), jnp.float32),
        ],
        compiler_params=pltpu.CompilerParams(needs_layout_passes=False),
    )(idx_flat)
    return out.reshape(B, _NSUB, n)


def _attn_body(q_ref, k_ref, v_ref, h_ref, out_ref, *, inv_u):
    m = q_ref.shape[1]
    n = k_ref.shape[1]
    d = q_ref.shape[2]
    u = int(_C * math.log(m))
    u_pad = ((u + 7) // 8) * 8
    nr = m // _ROW_BLK
    neg = jnp.float32(-3e38)

    k_all = k_ref[0]
    cnts = jnp.sum(h_ref[0], axis=0, keepdims=True)
    w2 = cnts * jnp.float32(inv_u)
    msk = cnts > 0.0

    m_rows = []
    for r in range(nr):
        qb = q_ref[0, r * _ROW_BLK:(r + 1) * _ROW_BLK, :]
        s = lax.dot_general(qb, k_all, (((1,), (1,)), ((), ())),
                            preferred_element_type=jnp.float32)
        mmax = jnp.max(jnp.where(msk, s, neg), axis=1)
        msum = jnp.sum(s * w2, axis=1)
        m_rows.append(mmax - msum)
    m_row = jnp.concatenate(m_rows)[None, :]
    m_col = m_row.reshape(m, 1)

    j_row = lax.broadcasted_iota(jnp.int32, (1, m), 1)
    cnt_cols = []
    for r in range(nr):
        a = m_col[r * _ROW_BLK:(r + 1) * _ROW_BLK, :]
        i_col = (lax.broadcasted_iota(jnp.int32, (_ROW_BLK, 1), 0)
                 + jnp.int32(r * _ROW_BLK))
        beats = (m_row > a) | ((m_row == a) & (j_row < i_col))
        cnt_cols.append(jnp.sum(jnp.where(beats, 1.0, 0.0),
                                axis=1, keepdims=True))
    cnt_col = jnp.concatenate(cnt_cols, axis=0)
    cnt_row = cnt_col.reshape(1, m)

    rid = lax.broadcasted_iota(jnp.int32, (u_pad, 1), 0).astype(jnp.float32)
    onehot = jnp.where((cnt_row == rid) & (rid < u), 1.0, 0.0)
    qbar = lax.dot_general(onehot, q_ref[0], (((1,), (0,)), ((), ())),
                           precision=lax.Precision.HIGHEST,
                           preferred_element_type=jnp.float32)

    s2 = lax.dot_general(qbar, k_all, (((1,), (1,)), ((), ())),
                         preferred_element_type=jnp.float32)
    s2 = s2 * jnp.float32(1.0 / math.sqrt(d))
    s2 = s2 - jnp.max(s2, axis=1, keepdims=True)
    e = jnp.exp(s2)
    p = e / jnp.sum(e, axis=1, keepdims=True)
    s1 = lax.dot_general(p, v_ref[0], (((1,), (0,)), ((), ())),
                         preferred_element_type=jnp.float32)

    for r in range(nr):
        ob = onehot[:, r * _ROW_BLK:(r + 1) * _ROW_BLK]
        cb = lax.dot_general(ob, s1, (((0,), (0,)), ((), ())),
                             precision=lax.Precision.HIGHEST,
                             preferred_element_type=jnp.float32)
        memb = cnt_cols[r] < u
        vb = v_ref[0, r * _ROW_BLK:(r + 1) * _ROW_BLK, :]
        vmb = jnp.sum(vb, axis=1, keepdims=True) * jnp.float32(1.0 / d)
        out_ref[0, r * _ROW_BLK:(r + 1) * _ROW_BLK, :] = (
            jnp.where(memb, cb, vmb))


def kernel(Q, K, V, rng):
    B, m, d = Q.shape
    n = K.shape[1]
    U = int(m * math.log(n))
    UP = ((U + 8 * _NSUB - 1) // (8 * _NSUB)) * (8 * _NSUB)

    rng_batch = jax.random.split(rng, B)
    idx = jax.vmap(
        lambda r: jax.random.choice(r, n, shape=(U,), replace=True))(rng_batch)
    idx_pad = jnp.pad(idx.astype(jnp.int32), ((0, 0), (0, UP - U)),
                      constant_values=-1)
    hist = _sc_histogram(idx_pad.reshape(-1), B, n)

    out = pl.pallas_call(
        functools.partial(_attn_body, inv_u=1.0 / U),
        grid=(B,),
        in_specs=[
            pl.BlockSpec((1, m, d), lambda b: (b, 0, 0)),
            pl.BlockSpec((1, n, d), lambda b: (b, 0, 0)),
            pl.BlockSpec((1, n, d), lambda b: (b, 0, 0)),
            pl.BlockSpec((1, _NSUB, n), lambda b: (b, 0, 0)),
        ],
        out_specs=pl.BlockSpec((1, m, d), lambda b: (b, 0, 0)),
        out_shape=jax.ShapeDtypeStruct((B, m, d), jnp.float32),
    )(Q, K, V, hist)
    return out


# speedup vs baseline: 4.5524x; 1.0322x over previous
import functools
import math

import jax
import jax.numpy as jnp
from jax import lax
from jax.experimental import pallas as pl
from jax.experimental.pallas import tpu as pltpu
from jax.experimental.pallas import tpu_sc as plsc

_C = 10
_ROW_BLK = 128
_NSUB = 16


def _sc_hist_body(idx_hbm, out_hbm, idx_v, loc_v):
    c = lax.axis_index("c")
    s = lax.axis_index("s")
    n = loc_v.shape[0]
    chunk = idx_v.shape[0]

    zeros16 = jnp.zeros((16,), jnp.float32)
    for i in range(n // 16):
        loc_v[pl.ds(i * 16, 16)] = zeros16

    pltpu.sync_copy(idx_hbm.at[pl.ds(c * (chunk * _NSUB) + s * chunk, chunk)],
                    idx_v)

    ones = jnp.ones((16,), jnp.float32)
    for i in range(chunk // 16):
        iv = idx_v[pl.ds(i * 16, 16)]
        plsc.addupdate_scatter(loc_v, [jnp.maximum(iv, 0)], ones,
                               mask=iv >= 0)

    pltpu.sync_copy(loc_v, out_hbm.at[pl.ds((c * _NSUB + s) * n, n)])


def _sc_histogram(idx_flat, B, n):
    UP = idx_flat.shape[0] // B
    mesh = plsc.VectorSubcoreMesh(core_axis_name="c", subcore_axis_name="s")
    out = pl.kernel(
        _sc_hist_body,
        out_type=jax.ShapeDtypeStruct((B * _NSUB * n,), jnp.float32),
        mesh=mesh,
        scratch_types=[
            pltpu.VMEM((UP // _NSUB,), jnp.int32),
            pltpu.VMEM((n,), jnp.float32),
        ],
        compiler_params=pltpu.CompilerParams(needs_layout_passes=False),
    )(idx_flat)
    return out.reshape(B, _NSUB, n)


def _attn_body(q_ref, k_ref, v_ref, h_ref, out_ref, *, inv_u):
    m = q_ref.shape[1]
    n = k_ref.shape[1]
    d = q_ref.shape[2]
    u = int(_C * math.log(m))
    u_pad = ((u + 7) // 8) * 8
    nr = m // _ROW_BLK
    neg = jnp.float32(-3e38)

    k_all = k_ref[0]
    cnts = jnp.sum(h_ref[0], axis=0, keepdims=True)
    w2row = cnts * jnp.float32(inv_u)
    msk = cnts > 0.0

    def _fold(x, op):
        w = x.shape[1]
        while w > 128:
            w //= 2
            x = op(x[:, :w], x[:, w:2 * w])
        return x

    m_rows = []
    for r in range(nr):
        qb = q_ref[0, r * _ROW_BLK:(r + 1) * _ROW_BLK, :]
        s = lax.dot_general(qb, k_all, (((1,), (1,)), ((), ())),
                            preferred_element_type=jnp.float32)
        mmax = jnp.max(_fold(jnp.where(msk, s, neg), jnp.maximum), axis=1)
        msum = jnp.sum(_fold(s * w2row, jnp.add), axis=1)
        m_rows.append(mmax - msum)
    m_row = jnp.concatenate(m_rows)[None, :]
    m_col = m_row.reshape(m, 1)

    j_row = lax.broadcasted_iota(jnp.int32, (1, m), 1)
    cnt_cols = []
    for r in range(nr):
        a = m_col[r * _ROW_BLK:(r + 1) * _ROW_BLK, :]
        i_col = (lax.broadcasted_iota(jnp.int32, (_ROW_BLK, 1), 0)
                 + jnp.int32(r * _ROW_BLK))
        beats = (m_row > a) | ((m_row == a) & (j_row < i_col))
        cnt_cols.append(jnp.sum(_fold(jnp.where(beats, 1.0, 0.0), jnp.add),
                                axis=1, keepdims=True))
    cnt_col = jnp.concatenate(cnt_cols, axis=0)
    cnt_row = cnt_col.reshape(1, m)

    rid = lax.broadcasted_iota(jnp.int32, (u_pad, 1), 0).astype(jnp.float32)
    onehot = jnp.where((cnt_row == rid) & (rid < u), 1.0, 0.0)
    qbar = lax.dot_general(onehot, q_ref[0], (((1,), (0,)), ((), ())),
                           preferred_element_type=jnp.float32)

    s2 = lax.dot_general(qbar, k_all, (((1,), (1,)), ((), ())),
                         preferred_element_type=jnp.float32)
    s2 = s2 * jnp.float32(1.0 / math.sqrt(d))
    s2 = s2 - jnp.max(s2, axis=1, keepdims=True)
    e = jnp.exp(s2)
    p = e / jnp.sum(e, axis=1, keepdims=True)
    s1 = lax.dot_general(p, v_ref[0], (((1,), (0,)), ((), ())),
                         preferred_element_type=jnp.float32)

    for r in range(nr):
        ob = onehot[:, r * _ROW_BLK:(r + 1) * _ROW_BLK]
        cb = lax.dot_general(ob, s1, (((0,), (0,)), ((), ())),
                             precision=lax.Precision.HIGHEST,
                             preferred_element_type=jnp.float32)
        memb = cnt_cols[r] < u
        vb = v_ref[0, r * _ROW_BLK:(r + 1) * _ROW_BLK, :]
        vmb = jnp.sum(vb, axis=1, keepdims=True) * jnp.float32(1.0 / d)
        out_ref[0, r * _ROW_BLK:(r + 1) * _ROW_BLK, :] = (
            jnp.where(memb, cb, vmb))


def kernel(Q, K, V, rng):
    B, m, d = Q.shape
    n = K.shape[1]
    U = int(m * math.log(n))
    UP = ((U + 8 * _NSUB - 1) // (8 * _NSUB)) * (8 * _NSUB)

    rng_batch = jax.random.split(rng, B)
    idx = jax.vmap(
        lambda r: jax.random.choice(r, n, shape=(U,), replace=True))(rng_batch)
    idx_pad = jnp.pad(idx.astype(jnp.int32), ((0, 0), (0, UP - U)),
                      constant_values=-1)
    hist = _sc_histogram(idx_pad.reshape(-1), B, n)

    out = pl.pallas_call(
        functools.partial(_attn_body, inv_u=1.0 / U),
        grid=(B,),
        in_specs=[
            pl.BlockSpec((1, m, d), lambda b: (b, 0, 0)),
            pl.BlockSpec((1, n, d), lambda b: (b, 0, 0)),
            pl.BlockSpec((1, n, d), lambda b: (b, 0, 0)),
            pl.BlockSpec((1, _NSUB, n), lambda b: (b, 0, 0)),
        ],
        out_specs=pl.BlockSpec((1, m, d), lambda b: (b, 0, 0)),
        out_shape=jax.ShapeDtypeStruct((B, m, d), jnp.float32),
    )(Q, K, V, hist)
    return out
